# baseline (speedup 1.0000x reference)
import functools

import jax
import jax.numpy as jnp
from jax import lax
from jax.experimental import pallas as pl
from jax.experimental.pallas import tpu as pltpu
from jax.experimental.pallas import tpu_sc as plsc

N_NODES = 10000
N_EDGES = 320000
D = 128
C = 10
G = 64

NC, NS = 2, 16
NW = NC * NS
EPW = N_EDGES // NW
CHUNK = 80
NCH = EPW // CHUNK
IB = 25
NBK = NCH // IB
S_CHUNK = 100
S_NCH = EPW // S_CHUNK
S_IB = 10
S_NBK = S_NCH // S_IB
RPT = 624
RPT_EXTRA = N_NODES - NS * RPT
ZR = 16
GPT = 8
BR = 1000
NBLK = N_NODES // BR

_HIGH = jax.lax.Precision.HIGHEST

_mesh = plsc.VectorSubcoreMesh(core_axis_name="c", subcore_axis_name="s",
                               num_cores=NC, num_subcores=NS)


def _zero_vmem(buf, nrows):
    def _z(r, carry):
        for c in range(D // 16):
            buf[r, pl.ds(c * 16, 16)] = jnp.zeros((16,), jnp.float32)
        return carry
    lax.fori_loop(0, nrows, _z, 0)


@functools.partial(
    pl.kernel,
    out_type=jax.ShapeDtypeStruct((NC, N_NODES, D), jnp.float32),
    mesh=_mesh,
    scratch_types=[
        pltpu.VMEM((S_IB, S_CHUNK), jnp.int32),
        pltpu.VMEM((S_IB, S_CHUNK), jnp.int32),
        pltpu.VMEM((S_CHUNK, D), jnp.float32),
        pltpu.VMEM((S_CHUNK, D), jnp.float32),
        pltpu.VMEM((S_CHUNK, D), jnp.float32),
        pltpu.VMEM((ZR, D), jnp.float32),
        pltpu.VMEM_SHARED((N_NODES, D), jnp.float32),
        pltpu.SemaphoreType.DMA,
        pltpu.SemaphoreType.DMA,
        pltpu.SemaphoreType.DMA,
        pltpu.SemaphoreType.DMA,
        pltpu.SemaphoreType.DMA,
        pltpu.SemaphoreType.DMA,
    ],
)
def _sc_scatter(a_hbm, src_hbm, dst_hbm, out_hbm, srcv, dstv, rows0, rows1,
                rows2, zbuf, acc, gs0, gs1, gs2, ss0, ss1, ss2):
    cid = lax.axis_index("c")
    sid = lax.axis_index("s")
    wid = cid * NS + sid
    _zero_vmem(zbuf, ZR)
    for z in range(RPT // ZR):
        pltpu.sync_copy(zbuf, acc.at[pl.ds(sid * RPT + z * ZR, ZR)])

    @pl.when(sid == 0)
    def _():
        pltpu.sync_copy(zbuf.at[pl.ds(0, RPT_EXTRA)],
                        acc.at[pl.ds(NS * RPT, RPT_EXTRA)])
    plsc.subcore_barrier()

    def _blk(b, carry):
        pltpu.sync_copy(src_hbm.at[wid, b], srcv)
        pltpu.sync_copy(dst_hbm.at[wid, b], dstv)
        pltpu.async_copy(a_hbm.at[srcv.at[0]], rows0, gs0)
        pltpu.async_copy(a_hbm.at[srcv.at[1]], rows1, gs1)

        def _round(j, carry2):
            i0 = 3 * j
            pltpu.make_async_copy(a_hbm.at[srcv.at[i0]], rows0, gs0).wait()
            pltpu.async_copy(rows0, acc.at[dstv.at[i0]], ss0, add=True)

            @pl.when(j > 0)
            def _():
                pltpu.make_async_copy(rows2, acc.at[dstv.at[i0]],
                                      ss2).wait()
            pltpu.async_copy(a_hbm.at[srcv.at[i0 + 2]], rows2, gs2)
            pltpu.make_async_copy(a_hbm.at[srcv.at[i0 + 1]], rows1,
                                  gs1).wait()
            pltpu.async_copy(rows1, acc.at[dstv.at[i0 + 1]], ss1, add=True)
            pltpu.make_async_copy(rows0, acc.at[dstv.at[i0]], ss0).wait()
            pltpu.async_copy(a_hbm.at[srcv.at[i0 + 3]], rows0, gs0)
            pltpu.make_async_copy(a_hbm.at[srcv.at[i0 + 2]], rows2,
                                  gs2).wait()
            pltpu.async_copy(rows2, acc.at[dstv.at[i0 + 2]], ss2, add=True)

            @pl.when(i0 + 4 < S_IB)
            def _():
                pltpu.make_async_copy(rows1, acc.at[dstv.at[i0 + 1]],
                                      ss1).wait()
                pltpu.async_copy(a_hbm.at[srcv.at[i0 + 4]], rows1, gs1)
            return carry2
        lax.fori_loop(0, S_IB // 3, _round, 0)
        pltpu.make_async_copy(a_hbm.at[srcv.at[S_IB - 1]], rows0, gs0).wait()
        pltpu.async_copy(rows0, acc.at[dstv.at[S_IB - 1]], ss0, add=True)
        pltpu.make_async_copy(rows0, acc.at[dstv.at[S_IB - 1]], ss0).wait()
        pltpu.make_async_copy(rows1, acc.at[dstv.at[S_IB - 3]], ss1).wait()
        pltpu.make_async_copy(rows2, acc.at[dstv.at[S_IB - 2]], ss2).wait()
        return carry
    lax.fori_loop(0, S_NBK, _blk, 0)
    plsc.subcore_barrier()
    pltpu.sync_copy(acc.at[pl.ds(sid * RPT, RPT)],
                    out_hbm.at[cid, pl.ds(sid * RPT, RPT)])

    @pl.when(sid == 0)
    def _():
        pltpu.sync_copy(acc.at[pl.ds(NS * RPT, RPT_EXTRA)],
                        out_hbm.at[cid, pl.ds(NS * RPT, RPT_EXTRA)])


CN = N_NODES * G
CPT = 39936
CEX = CN - NS * CPT
CZB = 4992


@functools.partial(
    pl.kernel,
    out_type=jax.ShapeDtypeStruct((NC * CN,), jnp.float32),
    mesh=_mesh,
    scratch_types=[
        pltpu.VMEM((IB, CHUNK), jnp.int32),
        pltpu.VMEM((IB, CHUNK), jnp.int32),
        pltpu.VMEM((IB, CHUNK), jnp.int32),
        pltpu.VMEM((N_NODES,), jnp.int32),
        pltpu.VMEM((CHUNK,), jnp.float32),
        pltpu.VMEM((CZB,), jnp.float32),
        pltpu.VMEM_SHARED((CN,), jnp.float32),
    ],
    compiler_params=pltpu.CompilerParams(needs_layout_passes=False),
)
def _sc_count(src_hbm, dst_hbm, batch_hbm, out_hbm, srcv, dstv, fbv, bvm,
              onesv, zbuf, acc):
    cid = lax.axis_index("c")
    sid = lax.axis_index("s")
    wid = cid * NS + sid
    pltpu.sync_copy(batch_hbm, bvm)
    for k in range(CHUNK // 16):
        onesv[pl.ds(k * 16, 16)] = jnp.ones((16,), jnp.float32)

    def _zf(r, carry):
        zbuf[pl.ds(r * 16, 16)] = jnp.zeros((16,), jnp.float32)
        return carry
    lax.fori_loop(0, CZB // 16, _zf, 0)
    for z in range(CPT // CZB):
        pltpu.sync_copy(zbuf, acc.at[pl.ds(sid * CPT + z * CZB, CZB)])

    @pl.when(sid == 0)
    def _():
        pltpu.sync_copy(zbuf.at[pl.ds(0, CEX)], acc.at[pl.ds(NS * CPT, CEX)])
    plsc.subcore_barrier()

    def _blk(b, carry):
        pltpu.sync_copy(src_hbm.at[wid, b], srcv)
        pltpu.sync_copy(dst_hbm.at[wid, b], dstv)

        def _chunk(i, carry2):
            for k in range(CHUNK // 16):
                sv = srcv[i, pl.ds(k * 16, 16)]
                dv = dstv[i, pl.ds(k * 16, 16)]
                gv = plsc.load_gather(bvm, [dv])
                fbv[i, pl.ds(k * 16, 16)] = sv * G + gv
            pltpu.sync_copy(onesv, acc.at[fbv.at[i]], add=True)
            return carry2
        lax.fori_loop(0, IB, _chunk, 0)
        return carry
    lax.fori_loop(0, NBK, _blk, 0)
    plsc.subcore_barrier()
    pltpu.sync_copy(acc.at[pl.ds(sid * CPT, CPT)],
                    out_hbm.at[pl.ds(cid * CN + sid * CPT, CPT)])

    @pl.when(sid == 0)
    def _():
        pltpu.sync_copy(acc.at[pl.ds(NS * CPT, CEX)],
                        out_hbm.at[pl.ds(cid * CN + NS * CPT, CEX)])


def _mm_in(x, Wa, Wb, b):
    def body(x_ref, wa_ref, wb_ref, b_ref, o1_ref, o2_ref):
        xb = x_ref[...]
        o1_ref[...] = jnp.dot(xb, wa_ref[...], precision=_HIGH,
                              preferred_element_type=jnp.float32)
        o2_ref[...] = jnp.dot(xb, wb_ref[...], precision=_HIGH,
                              preferred_element_type=jnp.float32) + b_ref[...]
    return pl.pallas_call(
        body,
        grid=(NBLK,),
        in_specs=[pl.BlockSpec((BR, D), lambda i: (i, 0)),
                  pl.BlockSpec((D, D), lambda i: (0, 0)),
                  pl.BlockSpec((D, D), lambda i: (0, 0)),
                  pl.BlockSpec((1, D), lambda i: (0, 0))],
        out_specs=[pl.BlockSpec((BR, D), lambda i: (i, 0)),
                   pl.BlockSpec((BR, D), lambda i: (i, 0))],
        out_shape=[jax.ShapeDtypeStruct((N_NODES, D), jnp.float32),
                   jax.ShapeDtypeStruct((N_NODES, D), jnp.float32)],
    )(x, Wa, Wb, b)


def _mm_mid(S, R, Wa, Wb, b):
    def body(s_ref, r_ref, wa_ref, wb_ref, b_ref, o1_ref, o2_ref):
        h = jnp.maximum(s_ref[0] + s_ref[1] + r_ref[...], 0.0)
        o1_ref[...] = jnp.dot(h, wa_ref[...], precision=_HIGH,
                              preferred_element_type=jnp.float32)
        o2_ref[...] = jnp.dot(h, wb_ref[...], precision=_HIGH,
                              preferred_element_type=jnp.float32) + b_ref[...]
    return pl.pallas_call(
        body,
        grid=(NBLK,),
        in_specs=[pl.BlockSpec((NC, BR, D), lambda i: (0, i, 0)),
                  pl.BlockSpec((BR, D), lambda i: (i, 0)),
                  pl.BlockSpec((D, D), lambda i: (0, 0)),
                  pl.BlockSpec((D, D), lambda i: (0, 0)),
                  pl.BlockSpec((1, D), lambda i: (0, 0))],
        out_specs=[pl.BlockSpec((BR, D), lambda i: (i, 0)),
                   pl.BlockSpec((BR, D), lambda i: (i, 0))],
        out_shape=[jax.ShapeDtypeStruct((N_NODES, D), jnp.float32),
                   jax.ShapeDtypeStruct((N_NODES, D), jnp.float32)],
    )(S, R, Wa, Wb, b)


def _mm_last_head(S, R, Cm, batch3, b3, Wr, Wt):
    def body(s_ref, r_ref, c_ref, bt_ref, b3_ref, wr_ref, wt_ref, o_ref,
             p_acc):
        i = pl.program_id(0)
        h = jnp.maximum(s_ref[0] + s_ref[1] + r_ref[...], 0.0)
        ids = bt_ref[0]
        oh = (ids == lax.broadcasted_iota(jnp.int32, (G, BR), 0)
              ).astype(jnp.float32)
        p2 = jnp.dot(oh, h, precision=_HIGH,
                     preferred_element_type=jnp.float32)
        cnt = jnp.sum(oh, axis=1, keepdims=True)
        cs = c_ref[0] + c_ref[1]
        p1 = lax.dot_general(cs, h,
                             dimension_numbers=(((0,), (0,)), ((), ())),
                             precision=_HIGH,
                             preferred_element_type=jnp.float32)
        pb = jnp.dot(p1, wr_ref[...], precision=_HIGH,
                     preferred_element_type=jnp.float32)
        pb = pb + jnp.dot(p2, wt_ref[...], precision=_HIGH,
                          preferred_element_type=jnp.float32)
        pb = pb + cnt * b3_ref[...]

        @pl.when(i == 0)
        def _():
            p_acc[...] = pb

        @pl.when(i > 0)
        def _():
            p_acc[...] += pb

        @pl.when(i == NBLK - 1)
        def _():
            p = p_acc[...]
            m = jnp.max(p, axis=1, keepdims=True)
            e = p - m
            lse = jnp.log(jnp.sum(jnp.exp(e), axis=1, keepdims=True))
            o_ref[...] = e - lse
    return pl.pallas_call(
        body,
        grid=(NBLK,),
        in_specs=[pl.BlockSpec((NC, BR, D), lambda i: (0, i, 0)),
                  pl.BlockSpec((BR, D), lambda i: (i, 0)),
                  pl.BlockSpec((NC, BR, G), lambda i: (0, i, 0)),
                  pl.BlockSpec((1, 1, BR), lambda i: (i, 0, 0)),
                  pl.BlockSpec((1, C), lambda i: (0, 0)),
                  pl.BlockSpec((D, C), lambda i: (0, 0)),
                  pl.BlockSpec((D, C), lambda i: (0, 0))],
        out_specs=pl.BlockSpec((G, C), lambda i: (0, 0)),
        out_shape=jax.ShapeDtypeStruct((G, C), jnp.float32),
        scratch_shapes=[pltpu.VMEM((G, C), jnp.float32)],
    )(S, R, Cm, batch3, b3, Wr, Wt)


def kernel(x, edge_index, batch, W_rel1, b_rel1, W_root1, W_rel2, b_rel2,
           W_root2, W_rel3, b_rel3, W_root3):
    src = edge_index[0].astype(jnp.int32).reshape(NW, NBK, IB, CHUNK)
    dst = edge_index[1].astype(jnp.int32).reshape(NW, NBK, IB, CHUNK)
    src_s = edge_index[0].astype(jnp.int32).reshape(NW, S_NBK, S_IB, S_CHUNK)
    dst_s = edge_index[1].astype(jnp.int32).reshape(NW, S_NBK, S_IB, S_CHUNK)
    batch_i = batch.astype(jnp.int32)
    batch3 = batch_i.reshape(NBLK, 1, BR)
    b1 = b_rel1.reshape(1, D)
    b2 = b_rel2.reshape(1, D)
    b3 = b_rel3.reshape(1, C)

    Cm = _sc_count(src, dst, batch_i).reshape(NC, N_NODES, G)
    A1, R1 = _mm_in(x, W_rel1, W_root1, b1)
    S1 = _sc_scatter(A1, src_s, dst_s)
    A2, R2 = _mm_mid(S1, R1, W_rel2, W_root2, b2)
    S2 = _sc_scatter(A2, src_s, dst_s)
    return _mm_last_head(S2, R2, Cm, batch3, b3, W_rel3, W_root3)


# speedup vs baseline: 1.0891x; 1.0891x over previous
import functools

import jax
import jax.numpy as jnp
from jax import lax
from jax.experimental import pallas as pl
from jax.experimental.pallas import tpu as pltpu
from jax.experimental.pallas import tpu_sc as plsc

N_NODES = 10000
N_EDGES = 320000
D = 128
C = 10
G = 64

NC, NS = 2, 16
NW = NC * NS
EPW = N_EDGES // NW
CHUNK = 80
NCH = EPW // CHUNK
IB = 25
NBK = NCH // IB
S_CHUNK = 100
S_NCH = EPW // S_CHUNK
S_IB = 25
S_NBK = S_NCH // S_IB
RPT = 624
RPT_EXTRA = N_NODES - NS * RPT
ZR = 16
GPT = 8
BR = 1000
NBLK = N_NODES // BR

_HIGH = jax.lax.Precision.HIGHEST

_mesh = plsc.VectorSubcoreMesh(core_axis_name="c", subcore_axis_name="s",
                               num_cores=NC, num_subcores=NS)


def _zero_vmem(buf, nrows):
    def _z(r, carry):
        for c in range(D // 16):
            buf[r, pl.ds(c * 16, 16)] = jnp.zeros((16,), jnp.float32)
        return carry
    lax.fori_loop(0, nrows, _z, 0)


@functools.partial(
    pl.kernel,
    out_type=jax.ShapeDtypeStruct((NC, N_NODES, D), jnp.float32),
    mesh=_mesh,
    scratch_types=[
        pltpu.VMEM((S_IB, S_CHUNK), jnp.int32),
        pltpu.VMEM((S_IB, S_CHUNK), jnp.int32),
        pltpu.VMEM((S_CHUNK, D), jnp.float32),
        pltpu.VMEM((S_CHUNK, D), jnp.float32),
        pltpu.VMEM((S_CHUNK, D), jnp.float32),
        pltpu.VMEM((ZR, D), jnp.float32),
        pltpu.VMEM_SHARED((N_NODES, D), jnp.float32),
        pltpu.SemaphoreType.DMA,
        pltpu.SemaphoreType.DMA,
        pltpu.SemaphoreType.DMA,
        pltpu.SemaphoreType.DMA,
        pltpu.SemaphoreType.DMA,
        pltpu.SemaphoreType.DMA,
    ],
)
def _sc_scatter(a_hbm, src_hbm, dst_hbm, out_hbm, srcv, dstv, rows0, rows1,
                rows2, zbuf, acc, gs0, gs1, gs2, ss0, ss1, ss2):
    cid = lax.axis_index("c")
    sid = lax.axis_index("s")
    wid = cid * NS + sid
    _zero_vmem(zbuf, ZR)
    for z in range(RPT // ZR):
        pltpu.sync_copy(zbuf, acc.at[pl.ds(sid * RPT + z * ZR, ZR)])

    @pl.when(sid == 0)
    def _():
        pltpu.sync_copy(zbuf.at[pl.ds(0, RPT_EXTRA)],
                        acc.at[pl.ds(NS * RPT, RPT_EXTRA)])
    plsc.subcore_barrier()

    def _blk(b, carry):
        pltpu.sync_copy(src_hbm.at[wid, b], srcv)
        pltpu.sync_copy(dst_hbm.at[wid, b], dstv)
        pltpu.async_copy(a_hbm.at[srcv.at[0]], rows0, gs0)
        pltpu.async_copy(a_hbm.at[srcv.at[1]], rows1, gs1)

        def _round(j, carry2):
            i0 = 3 * j
            pltpu.make_async_copy(a_hbm.at[srcv.at[i0]], rows0, gs0).wait()
            pltpu.async_copy(rows0, acc.at[dstv.at[i0]], ss0, add=True)

            @pl.when(j > 0)
            def _():
                pltpu.make_async_copy(rows2, acc.at[dstv.at[i0]],
                                      ss2).wait()
            pltpu.async_copy(a_hbm.at[srcv.at[i0 + 2]], rows2, gs2)
            pltpu.make_async_copy(a_hbm.at[srcv.at[i0 + 1]], rows1,
                                  gs1).wait()
            pltpu.async_copy(rows1, acc.at[dstv.at[i0 + 1]], ss1, add=True)
            pltpu.make_async_copy(rows0, acc.at[dstv.at[i0]], ss0).wait()
            pltpu.async_copy(a_hbm.at[srcv.at[i0 + 3]], rows0, gs0)
            pltpu.make_async_copy(a_hbm.at[srcv.at[i0 + 2]], rows2,
                                  gs2).wait()
            pltpu.async_copy(rows2, acc.at[dstv.at[i0 + 2]], ss2, add=True)

            @pl.when(i0 + 4 < S_IB)
            def _():
                pltpu.make_async_copy(rows1, acc.at[dstv.at[i0 + 1]],
                                      ss1).wait()
                pltpu.async_copy(a_hbm.at[srcv.at[i0 + 4]], rows1, gs1)
            return carry2
        lax.fori_loop(0, S_IB // 3, _round, 0)
        pltpu.make_async_copy(a_hbm.at[srcv.at[S_IB - 1]], rows0, gs0).wait()
        pltpu.async_copy(rows0, acc.at[dstv.at[S_IB - 1]], ss0, add=True)
        pltpu.make_async_copy(rows0, acc.at[dstv.at[S_IB - 1]], ss0).wait()
        pltpu.make_async_copy(rows1, acc.at[dstv.at[S_IB - 3]], ss1).wait()
        pltpu.make_async_copy(rows2, acc.at[dstv.at[S_IB - 2]], ss2).wait()
        return carry
    lax.fori_loop(0, S_NBK, _blk, 0)
    plsc.subcore_barrier()
    pltpu.sync_copy(acc.at[pl.ds(sid * RPT, RPT)],
                    out_hbm.at[cid, pl.ds(sid * RPT, RPT)])

    @pl.when(sid == 0)
    def _():
        pltpu.sync_copy(acc.at[pl.ds(NS * RPT, RPT_EXTRA)],
                        out_hbm.at[cid, pl.ds(NS * RPT, RPT_EXTRA)])


CN = N_NODES * G
CPT = 39936
CEX = CN - NS * CPT
CZB = 4992


@functools.partial(
    pl.kernel,
    out_type=jax.ShapeDtypeStruct((NC * CN,), jnp.float32),
    mesh=_mesh,
    scratch_types=[
        pltpu.VMEM((IB, CHUNK), jnp.int32),
        pltpu.VMEM((IB, CHUNK), jnp.int32),
        pltpu.VMEM((IB, CHUNK), jnp.int32),
        pltpu.VMEM((N_NODES,), jnp.int32),
        pltpu.VMEM((CHUNK,), jnp.float32),
        pltpu.VMEM((CZB,), jnp.float32),
        pltpu.VMEM_SHARED((CN,), jnp.float32),
    ],
    compiler_params=pltpu.CompilerParams(needs_layout_passes=False),
)
def _sc_count(src_hbm, dst_hbm, batch_hbm, out_hbm, srcv, dstv, fbv, bvm,
              onesv, zbuf, acc):
    cid = lax.axis_index("c")
    sid = lax.axis_index("s")
    wid = cid * NS + sid
    pltpu.sync_copy(batch_hbm, bvm)
    for k in range(CHUNK // 16):
        onesv[pl.ds(k * 16, 16)] = jnp.ones((16,), jnp.float32)

    def _zf(r, carry):
        zbuf[pl.ds(r * 16, 16)] = jnp.zeros((16,), jnp.float32)
        return carry
    lax.fori_loop(0, CZB // 16, _zf, 0)
    for z in range(CPT // CZB):
        pltpu.sync_copy(zbuf, acc.at[pl.ds(sid * CPT + z * CZB, CZB)])

    @pl.when(sid == 0)
    def _():
        pltpu.sync_copy(zbuf.at[pl.ds(0, CEX)], acc.at[pl.ds(NS * CPT, CEX)])
    plsc.subcore_barrier()

    def _blk(b, carry):
        pltpu.sync_copy(src_hbm.at[wid, b], srcv)
        pltpu.sync_copy(dst_hbm.at[wid, b], dstv)

        def _chunk(i, carry2):
            for k in range(CHUNK // 16):
                sv = srcv[i, pl.ds(k * 16, 16)]
                dv = dstv[i, pl.ds(k * 16, 16)]
                gv = plsc.load_gather(bvm, [dv])
                fbv[i, pl.ds(k * 16, 16)] = sv * G + gv
            pltpu.sync_copy(onesv, acc.at[fbv.at[i]], add=True)
            return carry2
        lax.fori_loop(0, IB, _chunk, 0)
        return carry
    lax.fori_loop(0, NBK, _blk, 0)
    plsc.subcore_barrier()
    pltpu.sync_copy(acc.at[pl.ds(sid * CPT, CPT)],
                    out_hbm.at[pl.ds(cid * CN + sid * CPT, CPT)])

    @pl.when(sid == 0)
    def _():
        pltpu.sync_copy(acc.at[pl.ds(NS * CPT, CEX)],
                        out_hbm.at[pl.ds(cid * CN + NS * CPT, CEX)])


def _mm_in(x, Wa, Wb, b):
    def body(x_ref, wa_ref, wb_ref, b_ref, o1_ref, o2_ref):
        xb = x_ref[...]
        o1_ref[...] = jnp.dot(xb, wa_ref[...], precision=_HIGH,
                              preferred_element_type=jnp.float32)
        o2_ref[...] = jnp.dot(xb, wb_ref[...], precision=_HIGH,
                              preferred_element_type=jnp.float32) + b_ref[...]
    return pl.pallas_call(
        body,
        grid=(NBLK,),
        in_specs=[pl.BlockSpec((BR, D), lambda i: (i, 0)),
                  pl.BlockSpec((D, D), lambda i: (0, 0)),
                  pl.BlockSpec((D, D), lambda i: (0, 0)),
                  pl.BlockSpec((1, D), lambda i: (0, 0))],
        out_specs=[pl.BlockSpec((BR, D), lambda i: (i, 0)),
                   pl.BlockSpec((BR, D), lambda i: (i, 0))],
        out_shape=[jax.ShapeDtypeStruct((N_NODES, D), jnp.float32),
                   jax.ShapeDtypeStruct((N_NODES, D), jnp.float32)],
    )(x, Wa, Wb, b)


def _mm_mid(S, R, Wa, Wb, b):
    def body(s_ref, r_ref, wa_ref, wb_ref, b_ref, o1_ref, o2_ref):
        h = jnp.maximum(s_ref[0] + s_ref[1] + r_ref[...], 0.0)
        o1_ref[...] = jnp.dot(h, wa_ref[...], precision=_HIGH,
                              preferred_element_type=jnp.float32)
        o2_ref[...] = jnp.dot(h, wb_ref[...], precision=_HIGH,
                              preferred_element_type=jnp.float32) + b_ref[...]
    return pl.pallas_call(
        body,
        grid=(NBLK,),
        in_specs=[pl.BlockSpec((NC, BR, D), lambda i: (0, i, 0)),
                  pl.BlockSpec((BR, D), lambda i: (i, 0)),
                  pl.BlockSpec((D, D), lambda i: (0, 0)),
                  pl.BlockSpec((D, D), lambda i: (0, 0)),
                  pl.BlockSpec((1, D), lambda i: (0, 0))],
        out_specs=[pl.BlockSpec((BR, D), lambda i: (i, 0)),
                   pl.BlockSpec((BR, D), lambda i: (i, 0))],
        out_shape=[jax.ShapeDtypeStruct((N_NODES, D), jnp.float32),
                   jax.ShapeDtypeStruct((N_NODES, D), jnp.float32)],
    )(S, R, Wa, Wb, b)


def _mm_last_head(S, R, Cm, batch3, b3, Wr, Wt):
    def body(s_ref, r_ref, c_ref, bt_ref, b3_ref, wr_ref, wt_ref, o_ref,
             p_acc):
        i = pl.program_id(0)
        h = jnp.maximum(s_ref[0] + s_ref[1] + r_ref[...], 0.0)
        ids = bt_ref[0]
        oh = (ids == lax.broadcasted_iota(jnp.int32, (G, BR), 0)
              ).astype(jnp.float32)
        p2 = jnp.dot(oh, h, precision=_HIGH,
                     preferred_element_type=jnp.float32)
        cnt = jnp.sum(oh, axis=1, keepdims=True)
        cs = c_ref[0] + c_ref[1]
        p1 = lax.dot_general(cs, h,
                             dimension_numbers=(((0,), (0,)), ((), ())),
                             precision=_HIGH,
                             preferred_element_type=jnp.float32)
        pb = jnp.dot(p1, wr_ref[...], precision=_HIGH,
                     preferred_element_type=jnp.float32)
        pb = pb + jnp.dot(p2, wt_ref[...], precision=_HIGH,
                          preferred_element_type=jnp.float32)
        pb = pb + cnt * b3_ref[...]

        @pl.when(i == 0)
        def _():
            p_acc[...] = pb

        @pl.when(i > 0)
        def _():
            p_acc[...] += pb

        @pl.when(i == NBLK - 1)
        def _():
            p = p_acc[...]
            m = jnp.max(p, axis=1, keepdims=True)
            e = p - m
            lse = jnp.log(jnp.sum(jnp.exp(e), axis=1, keepdims=True))
            o_ref[...] = e - lse
    return pl.pallas_call(
        body,
        grid=(NBLK,),
        in_specs=[pl.BlockSpec((NC, BR, D), lambda i: (0, i, 0)),
                  pl.BlockSpec((BR, D), lambda i: (i, 0)),
                  pl.BlockSpec((NC, BR, G), lambda i: (0, i, 0)),
                  pl.BlockSpec((1, 1, BR), lambda i: (i, 0, 0)),
                  pl.BlockSpec((1, C), lambda i: (0, 0)),
                  pl.BlockSpec((D, C), lambda i: (0, 0)),
                  pl.BlockSpec((D, C), lambda i: (0, 0))],
        out_specs=pl.BlockSpec((G, C), lambda i: (0, 0)),
        out_shape=jax.ShapeDtypeStruct((G, C), jnp.float32),
        scratch_shapes=[pltpu.VMEM((G, C), jnp.float32)],
    )(S, R, Cm, batch3, b3, Wr, Wt)


def kernel(x, edge_index, batch, W_rel1, b_rel1, W_root1, W_rel2, b_rel2,
           W_root2, W_rel3, b_rel3, W_root3):
    src = edge_index[0].astype(jnp.int32).reshape(NW, NBK, IB, CHUNK)
    dst = edge_index[1].astype(jnp.int32).reshape(NW, NBK, IB, CHUNK)
    src_s = edge_index[0].astype(jnp.int32).reshape(NW, S_NBK, S_IB, S_CHUNK)
    dst_s = edge_index[1].astype(jnp.int32).reshape(NW, S_NBK, S_IB, S_CHUNK)
    batch_i = batch.astype(jnp.int32)
    batch3 = batch_i.reshape(NBLK, 1, BR)
    b1 = b_rel1.reshape(1, D)
    b2 = b_rel2.reshape(1, D)
    b3 = b_rel3.reshape(1, C)

    Cm = _sc_count(src, dst, batch_i).reshape(NC, N_NODES, G)
    A1, R1 = _mm_in(x, W_rel1, W_root1, b1)
    S1 = _sc_scatter(A1, src_s, dst_s)
    A2, R2 = _mm_mid(S1, R1, W_rel2, W_root2, b2)
    S2 = _sc_scatter(A2, src_s, dst_s)
    return _mm_last_head(S2, R2, Cm, batch3, b3, W_rel3, W_root3)


# speedup vs baseline: 1.1168x; 1.0255x over previous
import functools

import jax
import jax.numpy as jnp
from jax import lax
from jax.experimental import pallas as pl
from jax.experimental.pallas import tpu as pltpu
from jax.experimental.pallas import tpu_sc as plsc

N_NODES = 10000
N_EDGES = 320000
D = 128
C = 10
G = 64

NC, NS = 2, 16
NW = NC * NS
EPW = N_EDGES // NW
CHUNK = 80
NCH = EPW // CHUNK
IB = 25
NBK = NCH // IB
S_CHUNK = 100
S_NCH = EPW // S_CHUNK
S_IB = 25
S_NBK = S_NCH // S_IB
S_BLK = S_IB * S_CHUNK
C_BLK = IB * CHUNK
RPT = 624
RPT_EXTRA = N_NODES - NS * RPT
ZR = 16
BR = 2000
NBLK = N_NODES // BR

_HIGH = jax.lax.Precision.HIGHEST

_mesh = plsc.VectorSubcoreMesh(core_axis_name="c", subcore_axis_name="s",
                               num_cores=NC, num_subcores=NS)


def _zero_vmem(buf, nrows):
    def _z(r, carry):
        for c in range(D // 16):
            buf[r, pl.ds(c * 16, 16)] = jnp.zeros((16,), jnp.float32)
        return carry
    lax.fori_loop(0, nrows, _z, 0)


@functools.partial(
    pl.kernel,
    out_type=jax.ShapeDtypeStruct((NC, N_NODES, D), jnp.float32),
    mesh=_mesh,
    scratch_types=[
        pltpu.VMEM((S_IB, S_CHUNK), jnp.int32),
        pltpu.VMEM((S_IB, S_CHUNK), jnp.int32),
        pltpu.VMEM((S_CHUNK, D), jnp.float32),
        pltpu.VMEM((S_CHUNK, D), jnp.float32),
        pltpu.VMEM((S_CHUNK, D), jnp.float32),
        pltpu.VMEM((ZR, D), jnp.float32),
        pltpu.VMEM_SHARED((N_NODES, D), jnp.float32),
        pltpu.SemaphoreType.DMA,
        pltpu.SemaphoreType.DMA,
        pltpu.SemaphoreType.DMA,
        pltpu.SemaphoreType.DMA,
        pltpu.SemaphoreType.DMA,
        pltpu.SemaphoreType.DMA,
    ],
)
def _sc_scatter(a_hbm, src_hbm, dst_hbm, out_hbm, srcv, dstv, rows0, rows1,
                rows2, zbuf, acc, gs0, gs1, gs2, ss0, ss1, ss2):
    cid = lax.axis_index("c")
    sid = lax.axis_index("s")
    wid = cid * NS + sid
    _zero_vmem(zbuf, ZR)
    for z in range(RPT // ZR):
        pltpu.sync_copy(zbuf, acc.at[pl.ds(sid * RPT + z * ZR, ZR)])

    @pl.when(sid == 0)
    def _():
        pltpu.sync_copy(zbuf.at[pl.ds(0, RPT_EXTRA)],
                        acc.at[pl.ds(NS * RPT, RPT_EXTRA)])
    plsc.subcore_barrier()

    def _c(i):
        return i

    def _blk(b, carry):
        pltpu.sync_copy(src_hbm.at[wid, b], srcv)
        pltpu.sync_copy(dst_hbm.at[wid, b], dstv)
        pltpu.async_copy(a_hbm.at[srcv.at[_c(0)]], rows0, gs0)
        pltpu.async_copy(a_hbm.at[srcv.at[_c(1)]], rows1, gs1)

        def _round(j, carry2):
            i0 = 3 * j
            pltpu.make_async_copy(a_hbm.at[srcv.at[_c(i0)]], rows0, gs0).wait()
            pltpu.async_copy(rows0, acc.at[dstv.at[_c(i0)]], ss0, add=True)

            @pl.when(j > 0)
            def _():
                pltpu.make_async_copy(rows2, acc.at[dstv.at[_c(i0)]],
                                      ss2).wait()
            pltpu.async_copy(a_hbm.at[srcv.at[_c(i0 + 2)]], rows2, gs2)
            pltpu.make_async_copy(a_hbm.at[srcv.at[_c(i0 + 1)]], rows1,
                                  gs1).wait()
            pltpu.async_copy(rows1, acc.at[dstv.at[_c(i0 + 1)]], ss1, add=True)
            pltpu.make_async_copy(rows0, acc.at[dstv.at[_c(i0)]], ss0).wait()
            pltpu.async_copy(a_hbm.at[srcv.at[_c(i0 + 3)]], rows0, gs0)
            pltpu.make_async_copy(a_hbm.at[srcv.at[_c(i0 + 2)]], rows2,
                                  gs2).wait()
            pltpu.async_copy(rows2, acc.at[dstv.at[_c(i0 + 2)]], ss2, add=True)

            @pl.when(i0 + 4 < S_IB)
            def _():
                pltpu.make_async_copy(rows1, acc.at[dstv.at[_c(i0 + 1)]],
                                      ss1).wait()
                pltpu.async_copy(a_hbm.at[srcv.at[_c(i0 + 4)]], rows1, gs1)
            return carry2
        lax.fori_loop(0, S_IB // 3, _round, 0)
        pltpu.make_async_copy(a_hbm.at[srcv.at[_c(S_IB - 1)]], rows0,
                              gs0).wait()
        pltpu.async_copy(rows0, acc.at[dstv.at[_c(S_IB - 1)]], ss0, add=True)
        pltpu.make_async_copy(rows0, acc.at[dstv.at[_c(S_IB - 1)]], ss0).wait()
        pltpu.make_async_copy(rows1, acc.at[dstv.at[_c(S_IB - 3)]], ss1).wait()
        pltpu.make_async_copy(rows2, acc.at[dstv.at[_c(S_IB - 2)]], ss2).wait()
        return carry
    lax.fori_loop(0, S_NBK, _blk, 0)
    plsc.subcore_barrier()
    pltpu.sync_copy(acc.at[pl.ds(sid * RPT, RPT)],
                    out_hbm.at[cid, pl.ds(sid * RPT, RPT)])

    @pl.when(sid == 0)
    def _():
        pltpu.sync_copy(acc.at[pl.ds(NS * RPT, RPT_EXTRA)],
                        out_hbm.at[cid, pl.ds(NS * RPT, RPT_EXTRA)])


CN = N_NODES * G
CPT = 39936
CEX = CN - NS * CPT
CZB = 4992


@functools.partial(
    pl.kernel,
    out_type=jax.ShapeDtypeStruct((NC * CN,), jnp.float32),
    mesh=_mesh,
    scratch_types=[
        pltpu.VMEM((IB, CHUNK), jnp.int32),
        pltpu.VMEM((IB, CHUNK), jnp.int32),
        pltpu.VMEM((IB, CHUNK), jnp.int32),
        pltpu.VMEM((N_NODES,), jnp.int32),
        pltpu.VMEM((CHUNK,), jnp.float32),
        pltpu.VMEM((CZB,), jnp.float32),
        pltpu.VMEM_SHARED((CN,), jnp.float32),
    ],
    compiler_params=pltpu.CompilerParams(needs_layout_passes=False),
)
def _sc_count(src_hbm, dst_hbm, batch_hbm, out_hbm, srcv, dstv, fbv, bvm,
              onesv, zbuf, acc):
    cid = lax.axis_index("c")
    sid = lax.axis_index("s")
    wid = cid * NS + sid
    pltpu.sync_copy(batch_hbm, bvm)
    for k in range(CHUNK // 16):
        onesv[pl.ds(k * 16, 16)] = jnp.ones((16,), jnp.float32)

    def _zf(r, carry):
        zbuf[pl.ds(r * 16, 16)] = jnp.zeros((16,), jnp.float32)
        return carry
    lax.fori_loop(0, CZB // 16, _zf, 0)
    for z in range(CPT // CZB):
        pltpu.sync_copy(zbuf, acc.at[pl.ds(sid * CPT + z * CZB, CZB)])

    @pl.when(sid == 0)
    def _():
        pltpu.sync_copy(zbuf.at[pl.ds(0, CEX)], acc.at[pl.ds(NS * CPT, CEX)])
    plsc.subcore_barrier()

    def _blk(b, carry):
        pltpu.sync_copy(src_hbm.at[wid, b], srcv)
        pltpu.sync_copy(dst_hbm.at[wid, b], dstv)

        def _chunk(i, carry2):
            for k in range(CHUNK // 16):
                o = pl.ds(k * 16, 16)
                sv = srcv[i, o]
                dv = dstv[i, o]
                gv = plsc.load_gather(bvm, [dv])
                fbv[i, o] = sv * G + gv
            pltpu.sync_copy(onesv, acc.at[fbv.at[i]], add=True)
            return carry2
        lax.fori_loop(0, IB, _chunk, 0)
        return carry
    lax.fori_loop(0, NBK, _blk, 0)
    plsc.subcore_barrier()
    pltpu.sync_copy(acc.at[pl.ds(sid * CPT, CPT)],
                    out_hbm.at[pl.ds(cid * CN + sid * CPT, CPT)])

    @pl.when(sid == 0)
    def _():
        pltpu.sync_copy(acc.at[pl.ds(NS * CPT, CEX)],
                        out_hbm.at[pl.ds(cid * CN + NS * CPT, CEX)])


def _mm_in(x, Wa, Wb, b):
    def body(x_ref, wa_ref, wb_ref, b_ref, o1_ref, o2_ref):
        xb = x_ref[...]
        o1_ref[...] = jnp.dot(xb, wa_ref[...], precision=_HIGH,
                              preferred_element_type=jnp.float32)
        o2_ref[...] = jnp.dot(xb, wb_ref[...], precision=_HIGH,
                              preferred_element_type=jnp.float32) + b_ref[...]
    return pl.pallas_call(
        body,
        grid=(NBLK,),
        in_specs=[pl.BlockSpec((BR, D), lambda i: (i, 0)),
                  pl.BlockSpec((D, D), lambda i: (0, 0)),
                  pl.BlockSpec((D, D), lambda i: (0, 0)),
                  pl.BlockSpec((1, D), lambda i: (0, 0))],
        out_specs=[pl.BlockSpec((BR, D), lambda i: (i, 0)),
                   pl.BlockSpec((BR, D), lambda i: (i, 0))],
        out_shape=[jax.ShapeDtypeStruct((N_NODES, D), jnp.float32),
                   jax.ShapeDtypeStruct((N_NODES, D), jnp.float32)],
    )(x, Wa, Wb, b)


def _mm_mid(S, R, Wa, Wb, b):
    def body(s_ref, r_ref, wa_ref, wb_ref, b_ref, o1_ref, o2_ref):
        h = jnp.maximum(s_ref[0] + s_ref[1] + r_ref[...], 0.0)
        o1_ref[...] = jnp.dot(h, wa_ref[...], precision=_HIGH,
                              preferred_element_type=jnp.float32)
        o2_ref[...] = jnp.dot(h, wb_ref[...], precision=_HIGH,
                              preferred_element_type=jnp.float32) + b_ref[...]
    return pl.pallas_call(
        body,
        grid=(NBLK,),
        in_specs=[pl.BlockSpec((NC, BR, D), lambda i: (0, i, 0)),
                  pl.BlockSpec((BR, D), lambda i: (i, 0)),
                  pl.BlockSpec((D, D), lambda i: (0, 0)),
                  pl.BlockSpec((D, D), lambda i: (0, 0)),
                  pl.BlockSpec((1, D), lambda i: (0, 0))],
        out_specs=[pl.BlockSpec((BR, D), lambda i: (i, 0)),
                   pl.BlockSpec((BR, D), lambda i: (i, 0))],
        out_shape=[jax.ShapeDtypeStruct((N_NODES, D), jnp.float32),
                   jax.ShapeDtypeStruct((N_NODES, D), jnp.float32)],
    )(S, R, Wa, Wb, b)


def _mm_last_head(S, R, Cm, batch3, b3, Wr, Wt):
    def body(s_ref, r_ref, c_ref, bt_ref, b3_ref, wr_ref, wt_ref, o_ref,
             p_acc):
        i = pl.program_id(0)
        h = jnp.maximum(s_ref[0] + s_ref[1] + r_ref[...], 0.0)
        ids = bt_ref[0]
        oh = (ids == lax.broadcasted_iota(jnp.int32, (G, BR), 0)
              ).astype(jnp.float32)
        p2 = jnp.dot(oh, h, precision=_HIGH,
                     preferred_element_type=jnp.float32)
        cnt = jnp.sum(oh, axis=1, keepdims=True)
        cs = c_ref[0] + c_ref[1]
        p1 = lax.dot_general(cs, h,
                             dimension_numbers=(((0,), (0,)), ((), ())),
                             precision=_HIGH,
                             preferred_element_type=jnp.float32)
        pb = jnp.dot(p1, wr_ref[...], precision=_HIGH,
                     preferred_element_type=jnp.float32)
        pb = pb + jnp.dot(p2, wt_ref[...], precision=_HIGH,
                          preferred_element_type=jnp.float32)
        pb = pb + cnt * b3_ref[...]

        @pl.when(i == 0)
        def _():
            p_acc[...] = pb

        @pl.when(i > 0)
        def _():
            p_acc[...] += pb

        @pl.when(i == NBLK - 1)
        def _():
            p = p_acc[...]
            m = jnp.max(p, axis=1, keepdims=True)
            e = p - m
            lse = jnp.log(jnp.sum(jnp.exp(e), axis=1, keepdims=True))
            o_ref[...] = e - lse
    return pl.pallas_call(
        body,
        grid=(NBLK,),
        in_specs=[pl.BlockSpec((NC, BR, D), lambda i: (0, i, 0)),
                  pl.BlockSpec((BR, D), lambda i: (i, 0)),
                  pl.BlockSpec((NC, BR, G), lambda i: (0, i, 0)),
                  pl.BlockSpec((1, 1, BR), lambda i: (i, 0, 0)),
                  pl.BlockSpec((1, C), lambda i: (0, 0)),
                  pl.BlockSpec((D, C), lambda i: (0, 0)),
                  pl.BlockSpec((D, C), lambda i: (0, 0))],
        out_specs=pl.BlockSpec((G, C), lambda i: (0, 0)),
        out_shape=jax.ShapeDtypeStruct((G, C), jnp.float32),
        scratch_shapes=[pltpu.VMEM((G, C), jnp.float32)],
    )(S, R, Cm, batch3, b3, Wr, Wt)


def kernel(x, edge_index, batch, W_rel1, b_rel1, W_root1, W_rel2, b_rel2,
           W_root2, W_rel3, b_rel3, W_root3):
    ei = edge_index.astype(jnp.int32)
    src = ei[0].reshape(NW, NBK, IB, CHUNK)
    dst = ei[1].reshape(NW, NBK, IB, CHUNK)
    src_s = ei[0].reshape(NW, S_NBK, S_IB, S_CHUNK)
    dst_s = ei[1].reshape(NW, S_NBK, S_IB, S_CHUNK)
    batch_i = batch.astype(jnp.int32)
    batch3 = batch_i.reshape(NBLK, 1, BR)
    b1 = b_rel1.reshape(1, D)
    b2 = b_rel2.reshape(1, D)
    b3 = b_rel3.reshape(1, C)

    Cm = _sc_count(src, dst, batch_i).reshape(NC, N_NODES, G)
    A1, R1 = _mm_in(x, W_rel1, W_root1, b1)
    S1 = _sc_scatter(A1, src_s, dst_s)
    A2, R2 = _mm_mid(S1, R1, W_rel2, W_root2, b2)
    S2 = _sc_scatter(A2, src_s, dst_s)
    return _mm_last_head(S2, R2, Cm, batch3, b3, W_rel3, W_root3)


# speedup vs baseline: 1.1789x; 1.0556x over previous
import functools

import jax
import jax.numpy as jnp
from jax import lax
from jax.experimental import pallas as pl
from jax.experimental.pallas import tpu as pltpu
from jax.experimental.pallas import tpu_sc as plsc

N_NODES = 10000
N_EDGES = 320000
D = 128
C = 10
G = 64

NC, NS = 2, 16
NW = NC * NS
EPW = N_EDGES // NW
CHUNK = 80
NCH = EPW // CHUNK
IB = 25
NBK = NCH // IB
S_CHUNK = 100
S_NCH = EPW // S_CHUNK
S_IB = 25
S_NBK = S_NCH // S_IB
S_BLK = S_IB * S_CHUNK
C_BLK = IB * CHUNK
RPT = 624
RPT_EXTRA = N_NODES - NS * RPT
ZR = 16
BR = 2000
NBLK = N_NODES // BR

_HIGH = jax.lax.Precision.HIGHEST

_mesh = plsc.VectorSubcoreMesh(core_axis_name="c", subcore_axis_name="s",
                               num_cores=NC, num_subcores=NS)


def _zero_vmem(buf, nrows):
    def _z(r, carry):
        for c in range(D // 16):
            buf[r, pl.ds(c * 16, 16)] = jnp.zeros((16,), jnp.float32)
        return carry
    lax.fori_loop(0, nrows, _z, 0)


@functools.partial(
    pl.kernel,
    out_type=jax.ShapeDtypeStruct((NC, N_NODES, D), jnp.float32),
    mesh=_mesh,
    scratch_types=[
        pltpu.VMEM((S_IB, S_CHUNK), jnp.int32),
        pltpu.VMEM((S_IB, S_CHUNK), jnp.int32),
        pltpu.VMEM((S_CHUNK, D), jnp.float32),
        pltpu.VMEM((S_CHUNK, D), jnp.float32),
        pltpu.VMEM((S_CHUNK, D), jnp.float32),
        pltpu.VMEM((ZR, D), jnp.float32),
        pltpu.VMEM_SHARED((N_NODES, D), jnp.float32),
        pltpu.SemaphoreType.DMA,
        pltpu.SemaphoreType.DMA,
        pltpu.SemaphoreType.DMA,
        pltpu.SemaphoreType.DMA,
        pltpu.SemaphoreType.DMA,
        pltpu.SemaphoreType.DMA,
    ],
)
def _sc_scatter(a_hbm, src_hbm, dst_hbm, out_hbm, srcv, dstv, rows0, rows1,
                rows2, zbuf, acc, gs0, gs1, gs2, ss0, ss1, ss2):
    cid = lax.axis_index("c")
    sid = lax.axis_index("s")
    wid = cid * NS + sid
    _zero_vmem(zbuf, ZR)
    for z in range(RPT // ZR):
        pltpu.sync_copy(zbuf, acc.at[pl.ds(sid * RPT + z * ZR, ZR)])

    @pl.when(sid == 0)
    def _():
        pltpu.sync_copy(zbuf.at[pl.ds(0, RPT_EXTRA)],
                        acc.at[pl.ds(NS * RPT, RPT_EXTRA)])
    plsc.subcore_barrier()

    def _c(i):
        return i

    def _blk(b, carry):
        pltpu.sync_copy(src_hbm.at[wid, b], srcv)
        pltpu.sync_copy(dst_hbm.at[wid, b], dstv)
        pltpu.async_copy(a_hbm.at[srcv.at[_c(0)]], rows0, gs0)
        pltpu.async_copy(a_hbm.at[srcv.at[_c(1)]], rows1, gs1)

        def _round(j, carry2):
            i0 = 3 * j
            pltpu.make_async_copy(a_hbm.at[srcv.at[_c(i0)]], rows0, gs0).wait()
            pltpu.async_copy(rows0, acc.at[dstv.at[_c(i0)]], ss0, add=True)

            @pl.when(j > 0)
            def _():
                pltpu.make_async_copy(rows2, acc.at[dstv.at[_c(i0)]],
                                      ss2).wait()
            pltpu.async_copy(a_hbm.at[srcv.at[_c(i0 + 2)]], rows2, gs2)
            pltpu.make_async_copy(a_hbm.at[srcv.at[_c(i0 + 1)]], rows1,
                                  gs1).wait()
            pltpu.async_copy(rows1, acc.at[dstv.at[_c(i0 + 1)]], ss1, add=True)
            pltpu.make_async_copy(rows0, acc.at[dstv.at[_c(i0)]], ss0).wait()
            pltpu.async_copy(a_hbm.at[srcv.at[_c(i0 + 3)]], rows0, gs0)
            pltpu.make_async_copy(a_hbm.at[srcv.at[_c(i0 + 2)]], rows2,
                                  gs2).wait()
            pltpu.async_copy(rows2, acc.at[dstv.at[_c(i0 + 2)]], ss2, add=True)

            @pl.when(i0 + 4 < S_IB)
            def _():
                pltpu.make_async_copy(rows1, acc.at[dstv.at[_c(i0 + 1)]],
                                      ss1).wait()
                pltpu.async_copy(a_hbm.at[srcv.at[_c(i0 + 4)]], rows1, gs1)
            return carry2
        lax.fori_loop(0, S_IB // 3, _round, 0)
        pltpu.make_async_copy(a_hbm.at[srcv.at[_c(S_IB - 1)]], rows0,
                              gs0).wait()
        pltpu.async_copy(rows0, acc.at[dstv.at[_c(S_IB - 1)]], ss0, add=True)
        pltpu.make_async_copy(rows0, acc.at[dstv.at[_c(S_IB - 1)]], ss0).wait()
        pltpu.make_async_copy(rows1, acc.at[dstv.at[_c(S_IB - 3)]], ss1).wait()
        pltpu.make_async_copy(rows2, acc.at[dstv.at[_c(S_IB - 2)]], ss2).wait()
        return carry
    lax.fori_loop(0, S_NBK, _blk, 0)
    plsc.subcore_barrier()
    pltpu.sync_copy(acc.at[pl.ds(sid * RPT, RPT)],
                    out_hbm.at[cid, pl.ds(sid * RPT, RPT)])

    @pl.when(sid == 0)
    def _():
        pltpu.sync_copy(acc.at[pl.ds(NS * RPT, RPT_EXTRA)],
                        out_hbm.at[cid, pl.ds(NS * RPT, RPT_EXTRA)])


CN = N_NODES * G
CPT = 39936
CEX = CN - NS * CPT
CZB = 4992


@functools.partial(
    pl.kernel,
    out_type=jax.ShapeDtypeStruct((NC * CN,), jnp.float32),
    mesh=_mesh,
    scratch_types=[
        pltpu.VMEM((IB, CHUNK), jnp.int32),
        pltpu.VMEM((IB, CHUNK), jnp.int32),
        pltpu.VMEM((IB, CHUNK), jnp.int32),
        pltpu.VMEM((IB, CHUNK), jnp.int32),
        pltpu.VMEM((IB, CHUNK), jnp.int32),
        pltpu.VMEM((IB, CHUNK), jnp.int32),
        pltpu.VMEM((N_NODES,), jnp.int32),
        pltpu.VMEM((CHUNK,), jnp.float32),
        pltpu.VMEM((CZB,), jnp.float32),
        pltpu.VMEM_SHARED((CN,), jnp.float32),
        pltpu.SemaphoreType.DMA,
        pltpu.SemaphoreType.DMA,
        pltpu.SemaphoreType.DMA,
        pltpu.SemaphoreType.DMA,
    ],
    compiler_params=pltpu.CompilerParams(needs_layout_passes=False),
)
def _sc_count(src_hbm, dst_hbm, batch_hbm, out_hbm, srcv0, srcv1, dstv0,
              dstv1, fbv0, fbv1, bvm, onesv, zbuf, acc, ls0, ls1, as0, as1):
    cid = lax.axis_index("c")
    sid = lax.axis_index("s")
    wid = cid * NS + sid
    srcv = (srcv0, srcv1)
    dstv = (dstv0, dstv1)
    fbv = (fbv0, fbv1)
    ls = (ls0, ls1)
    asem = (as0, as1)
    pltpu.async_copy(src_hbm.at[wid, 0], srcv[0], ls[0])
    pltpu.async_copy(dst_hbm.at[wid, 0], dstv[0], ls[0])
    pltpu.sync_copy(batch_hbm, bvm)
    for k in range(CHUNK // 16):
        onesv[pl.ds(k * 16, 16)] = jnp.ones((16,), jnp.float32)

    def _zf(r, carry):
        zbuf[pl.ds(r * 16, 16)] = jnp.zeros((16,), jnp.float32)
        return carry
    lax.fori_loop(0, CZB // 16, _zf, 0)
    for z in range(CPT // CZB):
        pltpu.sync_copy(zbuf, acc.at[pl.ds(sid * CPT + z * CZB, CZB)])

    @pl.when(sid == 0)
    def _():
        pltpu.sync_copy(zbuf.at[pl.ds(0, CEX)], acc.at[pl.ds(NS * CPT, CEX)])
    plsc.subcore_barrier()

    for b in range(NBK):
        p = b % 2
        if b + 1 < NBK:
            q = 1 - p
            pltpu.async_copy(src_hbm.at[wid, b + 1], srcv[q], ls[q])
            pltpu.async_copy(dst_hbm.at[wid, b + 1], dstv[q], ls[q])
        pltpu.make_async_copy(src_hbm.at[wid, b], srcv[p], ls[p]).wait()
        pltpu.make_async_copy(dst_hbm.at[wid, b], dstv[p], ls[p]).wait()
        if b >= 2:
            def _drain(i, carry):
                pltpu.make_async_copy(onesv, acc.at[fbv[p].at[i]],
                                      asem[p]).wait()
                return carry
            lax.fori_loop(0, IB, _drain, 0)

        def _chunk(i, carry2):
            for k in range(CHUNK // 16):
                o = pl.ds(k * 16, 16)
                sv = srcv[p][i, o]
                dv = dstv[p][i, o]
                gv = plsc.load_gather(bvm, [dv])
                fbv[p][i, o] = sv * G + gv
            pltpu.async_copy(onesv, acc.at[fbv[p].at[i]], asem[p], add=True)
            return carry2
        lax.fori_loop(0, IB, _chunk, 0)
    for b in (NBK - 2, NBK - 1):
        p = b % 2

        def _drain(i, carry):
            pltpu.make_async_copy(onesv, acc.at[fbv[p].at[i]],
                                  asem[p]).wait()
            return carry
        lax.fori_loop(0, IB, _drain, 0)
    plsc.subcore_barrier()
    pltpu.sync_copy(acc.at[pl.ds(sid * CPT, CPT)],
                    out_hbm.at[pl.ds(cid * CN + sid * CPT, CPT)])

    @pl.when(sid == 0)
    def _():
        pltpu.sync_copy(acc.at[pl.ds(NS * CPT, CEX)],
                        out_hbm.at[pl.ds(cid * CN + NS * CPT, CEX)])


def _mm_in(x, Wa, Wb, b):
    def body(x_ref, wa_ref, wb_ref, b_ref, o1_ref, o2_ref):
        xb = x_ref[...]
        o1_ref[...] = jnp.dot(xb, wa_ref[...], precision=_HIGH,
                              preferred_element_type=jnp.float32)
        o2_ref[...] = jnp.dot(xb, wb_ref[...], precision=_HIGH,
                              preferred_element_type=jnp.float32) + b_ref[...]
    return pl.pallas_call(
        body,
        grid=(NBLK,),
        in_specs=[pl.BlockSpec((BR, D), lambda i: (i, 0)),
                  pl.BlockSpec((D, D), lambda i: (0, 0)),
                  pl.BlockSpec((D, D), lambda i: (0, 0)),
                  pl.BlockSpec((1, D), lambda i: (0, 0))],
        out_specs=[pl.BlockSpec((BR, D), lambda i: (i, 0)),
                   pl.BlockSpec((BR, D), lambda i: (i, 0))],
        out_shape=[jax.ShapeDtypeStruct((N_NODES, D), jnp.float32),
                   jax.ShapeDtypeStruct((N_NODES, D), jnp.float32)],
    )(x, Wa, Wb, b)


def _mm_mid(S, R, Wa, Wb, b):
    def body(s_ref, r_ref, wa_ref, wb_ref, b_ref, o1_ref, o2_ref):
        h = jnp.maximum(s_ref[0] + s_ref[1] + r_ref[...], 0.0)
        o1_ref[...] = jnp.dot(h, wa_ref[...], precision=_HIGH,
                              preferred_element_type=jnp.float32)
        o2_ref[...] = jnp.dot(h, wb_ref[...], precision=_HIGH,
                              preferred_element_type=jnp.float32) + b_ref[...]
    return pl.pallas_call(
        body,
        grid=(NBLK,),
        in_specs=[pl.BlockSpec((NC, BR, D), lambda i: (0, i, 0)),
                  pl.BlockSpec((BR, D), lambda i: (i, 0)),
                  pl.BlockSpec((D, D), lambda i: (0, 0)),
                  pl.BlockSpec((D, D), lambda i: (0, 0)),
                  pl.BlockSpec((1, D), lambda i: (0, 0))],
        out_specs=[pl.BlockSpec((BR, D), lambda i: (i, 0)),
                   pl.BlockSpec((BR, D), lambda i: (i, 0))],
        out_shape=[jax.ShapeDtypeStruct((N_NODES, D), jnp.float32),
                   jax.ShapeDtypeStruct((N_NODES, D), jnp.float32)],
    )(S, R, Wa, Wb, b)


def _mm_last_head(S, R, Cm, batch3, b3, Wr, Wt):
    def body(s_ref, r_ref, c_ref, bt_ref, b3_ref, wr_ref, wt_ref, o_ref,
             p_acc):
        i = pl.program_id(0)
        h = jnp.maximum(s_ref[0] + s_ref[1] + r_ref[...], 0.0)
        ids = bt_ref[0]
        oh = (ids == lax.broadcasted_iota(jnp.int32, (G, BR), 0)
              ).astype(jnp.float32)
        p2 = jnp.dot(oh, h, precision=_HIGH,
                     preferred_element_type=jnp.float32)
        cnt = jnp.sum(oh, axis=1, keepdims=True)
        cs = c_ref[0] + c_ref[1]
        p1 = lax.dot_general(cs, h,
                             dimension_numbers=(((0,), (0,)), ((), ())),
                             precision=_HIGH,
                             preferred_element_type=jnp.float32)
        pb = jnp.dot(p1, wr_ref[...], precision=_HIGH,
                     preferred_element_type=jnp.float32)
        pb = pb + jnp.dot(p2, wt_ref[...], precision=_HIGH,
                          preferred_element_type=jnp.float32)
        pb = pb + cnt * b3_ref[...]

        @pl.when(i == 0)
        def _():
            p_acc[...] = pb

        @pl.when(i > 0)
        def _():
            p_acc[...] += pb

        @pl.when(i == NBLK - 1)
        def _():
            p = p_acc[...]
            m = jnp.max(p, axis=1, keepdims=True)
            e = p - m
            lse = jnp.log(jnp.sum(jnp.exp(e), axis=1, keepdims=True))
            o_ref[...] = e - lse
    return pl.pallas_call(
        body,
        grid=(NBLK,),
        in_specs=[pl.BlockSpec((NC, BR, D), lambda i: (0, i, 0)),
                  pl.BlockSpec((BR, D), lambda i: (i, 0)),
                  pl.BlockSpec((NC, BR, G), lambda i: (0, i, 0)),
                  pl.BlockSpec((1, 1, BR), lambda i: (i, 0, 0)),
                  pl.BlockSpec((1, C), lambda i: (0, 0)),
                  pl.BlockSpec((D, C), lambda i: (0, 0)),
                  pl.BlockSpec((D, C), lambda i: (0, 0))],
        out_specs=pl.BlockSpec((G, C), lambda i: (0, 0)),
        out_shape=jax.ShapeDtypeStruct((G, C), jnp.float32),
        scratch_shapes=[pltpu.VMEM((G, C), jnp.float32)],
    )(S, R, Cm, batch3, b3, Wr, Wt)


def kernel(x, edge_index, batch, W_rel1, b_rel1, W_root1, W_rel2, b_rel2,
           W_root2, W_rel3, b_rel3, W_root3):
    ei = edge_index.astype(jnp.int32)
    src = ei[0].reshape(NW, NBK, IB, CHUNK)
    dst = ei[1].reshape(NW, NBK, IB, CHUNK)
    src_s = ei[0].reshape(NW, S_NBK, S_IB, S_CHUNK)
    dst_s = ei[1].reshape(NW, S_NBK, S_IB, S_CHUNK)
    batch_i = batch.astype(jnp.int32)
    batch3 = batch_i.reshape(NBLK, 1, BR)
    b1 = b_rel1.reshape(1, D)
    b2 = b_rel2.reshape(1, D)
    b3 = b_rel3.reshape(1, C)

    Cm = _sc_count(src, dst, batch_i).reshape(NC, N_NODES, G)
    A1, R1 = _mm_in(x, W_rel1, W_root1, b1)
    S1 = _sc_scatter(A1, src_s, dst_s)
    A2, R2 = _mm_mid(S1, R1, W_rel2, W_root2, b2)
    S2 = _sc_scatter(A2, src_s, dst_s)
    return _mm_last_head(S2, R2, Cm, batch3, b3, W_rel3, W_root3)
